# baseline (speedup 1.0000x reference)
import jax
import jax.numpy as jnp
from jax.experimental import pallas as pl
from jax.experimental.pallas import tpu as pltpu

_T = 2048
_D = 1024
_E = 8
_HID = 1024
_OUT = 1024
_R = 8
_ER = _E * _R


def _dot(a, b, contract):
    return jax.lax.dot_general(
        a, b, (contract, ((), ())), preferred_element_type=jnp.float32
    )


def _moe_body(x_ref, w1_ref, acat_ref, b_ref, w2_ref, gate_ref, thr_ref,
              out_ref, wts_ref, xb_ref, lw_ref, call_ref, h_ref):
    s = pl.program_id(0)

    @pl.when(s == 0)
    def _compute_routing():
        x = x_ref[...]
        logits = _dot(x, gate_ref[...], (((1,), (1,))))
        m = jnp.max(logits, axis=-1, keepdims=True)
        p = jnp.exp(logits - m)
        probs = p / jnp.sum(p, axis=-1, keepdims=True)
        tl = _dot(x, thr_ref[...], (((1,), (1,))))
        thr = jax.nn.sigmoid(tl) * (1.0 / _E)
        w = jnp.maximum(probs - thr, 0.0)
        ssum = jnp.sum(w, axis=-1, keepdims=True)
        ssum = jnp.where(ssum == 0.0, 1.0, ssum)
        wts_ref[...] = w / ssum
        xb_ref[...] = x.astype(jnp.bfloat16)

    group = jax.lax.broadcasted_iota(jnp.int32, (_T, _ER), 1) // _R

    def _project(src, expert):
        lf = _dot(src, acat_ref[...].astype(jnp.bfloat16),
                  (((1,), (0,))))
        lw_ref[...] = jnp.where(group == expert, lf.astype(jnp.bfloat16),
                                lw_ref[...])

    @pl.when((s > 0) & (s <= _E))
    def _expert_step():
        _project(h_ref[(s - 1) % 2], jnp.clip(s - 2, 0, _E - 1))
        w1b = w1_ref[0].astype(jnp.bfloat16)
        h_ref[s % 2] = jnp.maximum(
            _dot(xb_ref[...], w1b, (((1,), (1,)))), 0.0
        ).astype(jnp.bfloat16)
        c = _dot(b_ref[0], w2_ref[0], (((1,), (1,))))
        call_ref[pl.ds((s - 1) * _R, _R), :] = c

    @pl.when(s == _E + 1)
    def _final():
        _project(h_ref[_E % 2], _E - 1)
        sel = (jax.lax.broadcasted_iota(jnp.int32, (_E, _ER), 0)
               == jax.lax.broadcasted_iota(jnp.int32, (_E, _ER), 1) // _R)
        wexp = _dot(wts_ref[...], sel.astype(jnp.float32), (((1,), (0,))))
        lws = (lw_ref[...].astype(jnp.float32) * wexp).astype(jnp.bfloat16)
        out_ref[...] = _dot(lws, call_ref[...].astype(jnp.bfloat16),
                            (((1,), (0,))))


def kernel(x, W1, lora_A, lora_B, W2, gate_w, thr_w):
    Bsz, Tlen, H = x.shape
    flat = x.reshape(Tlen, H)
    a_cat = jnp.transpose(lora_A, (1, 0, 2)).reshape(_HID, _ER)
    ecap = lambda s: jnp.clip(s - 1, 0, _E - 1)
    out = pl.pallas_call(
        _moe_body,
        grid=(_E + 2,),
        in_specs=[
            pl.BlockSpec((_T, _D), lambda s: (0, 0)),
            pl.BlockSpec((1, _HID, _D), lambda s: (ecap(s), 0, 0)),
            pl.BlockSpec((_HID, _ER), lambda s: (0, 0)),
            pl.BlockSpec((1, _R, _OUT), lambda s: (ecap(s), 0, 0)),
            pl.BlockSpec((1, _OUT, _HID), lambda s: (ecap(s), 0, 0)),
            pl.BlockSpec((_E, _D), lambda s: (0, 0)),
            pl.BlockSpec((1, _D), lambda s: (0, 0)),
        ],
        out_specs=pl.BlockSpec((_T, _OUT), lambda s: (0, 0)),
        out_shape=jax.ShapeDtypeStruct((_T, _OUT), jnp.float32),
        scratch_shapes=[
            pltpu.VMEM((_T, _E), jnp.float32),
            pltpu.VMEM((_T, _D), jnp.bfloat16),
            pltpu.VMEM((_T, _ER), jnp.bfloat16),
            pltpu.VMEM((_ER, _OUT), jnp.float32),
            pltpu.VMEM((2, _T, _HID), jnp.bfloat16),
        ],
        compiler_params=pltpu.CompilerParams(
            dimension_semantics=("arbitrary",),
            vmem_limit_bytes=100 * 1024 * 1024,
        ),
    )(flat, W1, a_cat, lora_B, W2, gate_w, thr_w)
    return out.reshape(Bsz, Tlen, _OUT)


# speedup vs baseline: 1.0339x; 1.0339x over previous
import jax
import jax.numpy as jnp
from jax.experimental import pallas as pl
from jax.experimental.pallas import tpu as pltpu

_T = 2048
_D = 1024
_E = 8
_HID = 1024
_OUT = 1024
_R = 8
_ER = _E * _R


def _dot(a, b, contract):
    return jax.lax.dot_general(
        a, b, (contract, ((), ())), preferred_element_type=jnp.float32
    )


def _moe_body(x_ref, w1_ref, acat_ref, b_ref, w2_ref, gate_ref, thr_ref,
              out_ref, wts_ref, xb_ref, lw_ref, call_ref, h_ref):
    s = pl.program_id(0)

    @pl.when(s == 0)
    def _compute_routing():
        x = x_ref[...]
        logits = _dot(x, gate_ref[...], (((1,), (1,))))
        m = jnp.max(logits, axis=-1, keepdims=True)
        p = jnp.exp(logits - m)
        probs = p / jnp.sum(p, axis=-1, keepdims=True)
        tl = _dot(x, thr_ref[...], (((1,), (1,))))
        thr = jax.nn.sigmoid(tl) * (1.0 / _E)
        w = jnp.maximum(probs - thr, 0.0)
        ssum = jnp.sum(w, axis=-1, keepdims=True)
        ssum = jnp.where(ssum == 0.0, 1.0, ssum)
        wts_ref[...] = w / ssum
        xb_ref[...] = x.astype(jnp.bfloat16)

    @pl.when(s > 1)
    def _project_prev():
        lf = _dot(h_ref[...], acat_ref[...].astype(jnp.bfloat16),
                  (((1,), (0,))))
        group = jax.lax.broadcasted_iota(jnp.int32, (_T, _ER), 1) // _R
        lw_ref[...] = jnp.where(group == s - 2, lf.astype(jnp.bfloat16),
                                lw_ref[...])

    @pl.when((s > 0) & (s <= _E))
    def _expert_matmul():
        w1b = w1_ref[0].astype(jnp.bfloat16)
        h_ref[...] = jnp.maximum(
            _dot(xb_ref[...], w1b, (((1,), (1,)))), 0.0
        ).astype(jnp.bfloat16)
        c = _dot(b_ref[0], w2_ref[0], (((1,), (1,))))
        call_ref[pl.ds((s - 1) * _R, _R), :] = c

    @pl.when(s == _E + 1)
    def _final():
        sel = (jax.lax.broadcasted_iota(jnp.int32, (_E, _ER), 0)
               == jax.lax.broadcasted_iota(jnp.int32, (_E, _ER), 1) // _R)
        wexp = _dot(wts_ref[...], sel.astype(jnp.float32), (((1,), (0,))))
        lws = (lw_ref[...].astype(jnp.float32) * wexp).astype(jnp.bfloat16)
        out_ref[...] = _dot(lws, call_ref[...].astype(jnp.bfloat16),
                            (((1,), (0,))))


def kernel(x, W1, lora_A, lora_B, W2, gate_w, thr_w):
    Bsz, Tlen, H = x.shape
    flat = x.reshape(Tlen, H)
    a_cat = jnp.transpose(lora_A, (1, 0, 2)).reshape(_HID, _ER)
    ecap = lambda s: jnp.clip(s - 1, 0, _E - 1)
    out = pl.pallas_call(
        _moe_body,
        grid=(_E + 2,),
        in_specs=[
            pl.BlockSpec((_T, _D), lambda s: (0, 0)),
            pl.BlockSpec((1, _HID, _D), lambda s: (ecap(s), 0, 0)),
            pl.BlockSpec((_HID, _ER), lambda s: (0, 0)),
            pl.BlockSpec((1, _R, _OUT), lambda s: (ecap(s), 0, 0)),
            pl.BlockSpec((1, _OUT, _HID), lambda s: (ecap(s), 0, 0)),
            pl.BlockSpec((_E, _D), lambda s: (0, 0)),
            pl.BlockSpec((1, _D), lambda s: (0, 0)),
        ],
        out_specs=pl.BlockSpec((_T, _OUT), lambda s: (0, 0)),
        out_shape=jax.ShapeDtypeStruct((_T, _OUT), jnp.float32),
        scratch_shapes=[
            pltpu.VMEM((_T, _E), jnp.float32),
            pltpu.VMEM((_T, _D), jnp.bfloat16),
            pltpu.VMEM((_T, _ER), jnp.bfloat16),
            pltpu.VMEM((_ER, _OUT), jnp.float32),
            pltpu.VMEM((_T, _HID), jnp.bfloat16),
        ],
        compiler_params=pltpu.CompilerParams(
            dimension_semantics=("arbitrary",),
            vmem_limit_bytes=100 * 1024 * 1024,
        ),
    )(flat, W1, a_cat, lora_B, W2, gate_w, thr_w)
    return out.reshape(Bsz, Tlen, _OUT)
